# Initial kernel scaffold; baseline (speedup 1.0000x reference)
import jax
import jax.numpy as jnp
from jax.experimental import pallas as pl


def kernel(tokenized_text, embedded_text, tokenizer_id, face_img_embeddings, W1, b1, W2, b2, celeb_mean, celeb_std):
    raise NotImplementedError("write your pallas kernel here")


# speedup vs baseline: 1.1612x; 1.1612x over previous
import jax
import jax.numpy as jnp
from jax.experimental import pallas as pl
from jax.experimental.pallas import tpu as pltpu

_LR_MUL = 0.1
_PLACEHOLDER = 265
_BA = 256
_BB = 8


def _mlp_body(s_ref, tok_ref, face_ref, w1_ref, b1_ref, w2_ref, b2_ref,
              cm_ref, cs_ref, text_ref, pos_ref):
    s = s_ref[0, 0]
    x = face_ref[...]
    nrm = jnp.sqrt(jnp.sum(x * x, axis=1, keepdims=True))
    x = x / jnp.maximum(nrm, 1e-12)
    h = jax.lax.dot_general(x, w1_ref[...], (((1,), (1,)), ((), ())),
                            preferred_element_type=jnp.float32)
    h = h * _LR_MUL + b1_ref[...] * _LR_MUL
    h = jnp.where(h > 0, h, h * 0.2)
    h = jax.lax.dot_general(h, w2_ref[...], (((1,), (1,)), ((), ())),
                            preferred_element_type=jnp.float32)
    h = h * _LR_MUL + b2_ref[...] * _LR_MUL
    h = jnp.where(h > 0, h, h * 0.2)
    text_ref[...] = (cm_ref[...] + h * cs_ref[...]) * s
    tok = tok_ref[...]
    seq = tok.shape[1]
    col = jax.lax.broadcasted_iota(jnp.int32, tok.shape, 1)
    pos_ref[...] = jnp.min(jnp.where(tok == _PLACEHOLDER, col, seq), axis=1,
                           keepdims=True)


def _stream_body(s_ref, pos_ref, text_ref, emb_ref, out_ref):
    s = s_ref[0, 0]
    bb, seq, d = emb_ref.shape
    for r in range(bb):
        p = pos_ref[r, 0]
        t0 = text_ref[pl.ds(r, 1), 0:d]
        t1 = text_ref[pl.ds(r, 1), d:2 * d]
        e = emb_ref[pl.ds(r, 1)]
        row = jax.lax.broadcasted_iota(jnp.int32, (1, seq, d), 1)
        out_ref[pl.ds(r, 1)] = jnp.where(
            row == p, t0[:, None, :],
            jnp.where(row == p + 1, t1[:, None, :], e * s))


def kernel(tokenized_text, embedded_text, tokenizer_id, face_img_embeddings,
           W1, b1, W2, b2, celeb_mean, celeb_std):
    B, S, D = embedded_text.shape
    H = W1.shape[0]
    V = W1.shape[1]
    s = jnp.asarray(tokenizer_id, embedded_text.dtype).reshape(1, 1)
    b1r = b1.reshape(1, H)
    b2r = b2.reshape(1, H)
    cm = celeb_mean.reshape(1, H)
    cs = celeb_std.reshape(1, H)

    text, pos = pl.pallas_call(
        _mlp_body,
        grid=(B // _BA,),
        in_specs=[
            pl.BlockSpec((1, 1), lambda i: (0, 0), memory_space=pltpu.SMEM),
            pl.BlockSpec((_BA, S), lambda i: (i, 0)),
            pl.BlockSpec((_BA, V), lambda i: (i, 0)),
            pl.BlockSpec((H, V), lambda i: (0, 0)),
            pl.BlockSpec((1, H), lambda i: (0, 0)),
            pl.BlockSpec((H, H), lambda i: (0, 0)),
            pl.BlockSpec((1, H), lambda i: (0, 0)),
            pl.BlockSpec((1, H), lambda i: (0, 0)),
            pl.BlockSpec((1, H), lambda i: (0, 0)),
        ],
        out_specs=[
            pl.BlockSpec((_BA, H), lambda i: (i, 0)),
            pl.BlockSpec((_BA, 1), lambda i: (i, 0)),
        ],
        out_shape=[
            jax.ShapeDtypeStruct((B, H), embedded_text.dtype),
            jax.ShapeDtypeStruct((B, 1), jnp.int32),
        ],
    )(s, tokenized_text, face_img_embeddings, W1, b1r, W2, b2r, cm, cs)

    return pl.pallas_call(
        _stream_body,
        grid=(B // _BB,),
        in_specs=[
            pl.BlockSpec((1, 1), lambda i: (0, 0), memory_space=pltpu.SMEM),
            pl.BlockSpec((_BB, 1), lambda i: (i, 0), memory_space=pltpu.SMEM),
            pl.BlockSpec((_BB, 2 * D), lambda i: (i, 0)),
            pl.BlockSpec((_BB, S, D), lambda i: (i, 0, 0)),
        ],
        out_specs=pl.BlockSpec((_BB, S, D), lambda i: (i, 0, 0)),
        out_shape=jax.ShapeDtypeStruct((B, S, D), embedded_text.dtype),
    )(s, pos, text, embedded_text)
